# Initial kernel scaffold; baseline (speedup 1.0000x reference)
import jax
import jax.numpy as jnp
from jax.experimental import pallas as pl


def kernel(x, edge_index, edge_attr, W1, root1, b1, W2, root2, b2):
    raise NotImplementedError("write your pallas kernel here")


# speedup vs baseline: 4.5817x; 4.5817x over previous
import functools

import jax
import jax.numpy as jnp
from jax import lax
from jax.experimental import pallas as pl
from jax.experimental.pallas import tpu as pltpu
from jax.experimental.pallas import tpu_sc as plsc

N = 10000
E = 160000
C = 128
R = 8
LANES = 16
NF = C // LANES
NS = 16
NC = 2
CH = 128
EPT = 10240
EP = EPT * NS
NCH = EPT // CH
KPAD = R * N
ACC_ROWS = 81920
RPT_Z = ACC_ROWS // NS
ZROWS = RPT_Z // 8
RPT = (R * N) // NS
BLK = 1000


def _sc_body(with_cnt, *refs):
    if with_cnt:
        (xb, srcp, dstp, typp, zeros_h, ones_h, s4, cnt3,
         src_v, kb, dstb, typb, rowb, onesb, zb, acc, sem) = refs
    else:
        (xb, srcp, dstp, typp, zeros_h, ones_h, s4,
         src_v, kb, dstb, typb, rowb, onesb, zb, acc, sem) = refs
        cnt3 = None

    c = lax.axis_index("c")
    s = lax.axis_index("s")
    e0 = s * EPT

    pltpu.sync_copy(srcp.at[pl.ds(e0, EPT)], src_v)
    pltpu.sync_copy(zeros_h, zb)
    pltpu.sync_copy(ones_h, onesb)

    def krow(j, carry):
        pltpu.sync_copy(dstp.at[pl.ds(e0 + j * CH, CH)], dstb)
        pltpu.sync_copy(typp.at[pl.ds(e0 + j * CH, CH)], typb)
        for u in range(CH // LANES):
            t16 = typb[pl.ds(u * LANES, LANES)]
            d16 = dstb[pl.ds(u * LANES, LANES)]
            kb[j, pl.ds(u * LANES, LANES)] = t16 * N + d16
        return carry
    lax.fori_loop(0, NCH, krow, 0)

    def zero_acc():
        def zbody(i, carry):
            pltpu.sync_copy(zb, acc.at[pl.ds(s * RPT_Z + i * ZROWS, ZROWS)])
            return carry
        lax.fori_loop(0, 8, zbody, 0)

    def scatter_pass(table):
        def chunk(j, carry):
            if table is not None:
                idx = src_v.at[pl.ds(j * CH, CH)]
                pltpu.async_copy(table.at[idx], rowb, sem).wait()
                rows = rowb
            else:
                rows = onesb
            pltpu.sync_copy(rows, acc.at[kb.at[j]], add=True)
            return carry
        lax.fori_loop(0, NCH, chunk, 0)

    def writeback(dst_ref, f):
        r = s // 2
        i0 = (s % 2) * RPT
        src_slice = acc.at[pl.ds(s * RPT, RPT)]
        if f is None:
            pltpu.sync_copy(src_slice, dst_ref.at[r, pl.ds(i0, RPT)])
        else:
            pltpu.sync_copy(src_slice, dst_ref.at[r, pl.ds(i0, RPT), f])

    for cv in range(NC):
        @pl.when(c == cv)
        def _passes(cv=cv):
            for p in range(NF // NC):
                f = cv * (NF // NC) + p
                zero_acc()
                plsc.subcore_barrier()
                scatter_pass(xb.at[f])
                plsc.subcore_barrier()
                writeback(s4, f)
                plsc.subcore_barrier()

    if with_cnt:
        @pl.when(c == 0)
        def _cnt_pass():
            zero_acc()
            plsc.subcore_barrier()
            scatter_pass(None)
            plsc.subcore_barrier()
            writeback(cnt3, None)


def _make_sc(with_cnt):
    out_types = [jax.ShapeDtypeStruct((R, N, NF, LANES), jnp.float32)]
    if with_cnt:
        out_types.append(jax.ShapeDtypeStruct((R, N, LANES), jnp.float32))
    scratch = [
        pltpu.VMEM((EPT,), jnp.int32),
        pltpu.VMEM((NCH, CH), jnp.int32),
        pltpu.VMEM((CH,), jnp.int32),
        pltpu.VMEM((CH,), jnp.int32),
        pltpu.VMEM((CH, LANES), jnp.float32),
        pltpu.VMEM((CH, LANES), jnp.float32),
        pltpu.VMEM((ZROWS, LANES), jnp.float32),
        pltpu.VMEM_SHARED((ACC_ROWS, LANES), jnp.float32),
        pltpu.SemaphoreType.DMA,
    ]
    mesh = plsc.VectorSubcoreMesh(core_axis_name="c", subcore_axis_name="s")
    return pl.kernel(
        functools.partial(_sc_body, with_cnt),
        out_type=tuple(out_types) if with_cnt else out_types[0],
        mesh=mesh,
        scratch_types=scratch,
        compiler_params=pltpu.CompilerParams(use_tc_tiling_on_sc=False),
    )


_make_sc = functools.lru_cache(maxsize=None)(_make_sc)


def _tc_body(relu, x_ref, s_ref, c_ref, w_ref, root_ref, b_ref, o_ref):
    acc = jnp.dot(x_ref[...], root_ref[...],
                  preferred_element_type=jnp.float32) + b_ref[...]
    for r in range(R):
        cr = c_ref[r][:, 0:1]
        inv = 1.0 / jnp.maximum(cr, 1.0)
        acc = acc + jnp.dot(s_ref[r] * inv, w_ref[r],
                            preferred_element_type=jnp.float32)
    o_ref[...] = jnp.maximum(acc, 0.0) if relu else acc


def _tc_combine(xin, S, cnt3, W, root, b, relu):
    return pl.pallas_call(
        functools.partial(_tc_body, relu),
        grid=(N // BLK,),
        in_specs=[
            pl.BlockSpec((BLK, C), lambda i: (i, 0)),
            pl.BlockSpec((R, BLK, C), lambda i: (0, i, 0)),
            pl.BlockSpec((R, BLK, LANES), lambda i: (0, i, 0)),
            pl.BlockSpec((R, C, C), lambda i: (0, 0, 0)),
            pl.BlockSpec((C, C), lambda i: (0, 0)),
            pl.BlockSpec((1, C), lambda i: (0, 0)),
        ],
        out_specs=pl.BlockSpec((BLK, C), lambda i: (i, 0)),
        out_shape=jax.ShapeDtypeStruct((N, C), jnp.float32),
    )(xin, S, cnt3, W, root, b.reshape(1, C))


def kernel(x, edge_index, edge_attr, W1, root1, b1, W2, root2, b2):
    src = edge_index[0].astype(jnp.int32)
    dst = edge_index[1].astype(jnp.int32)
    typ = edge_attr.astype(jnp.int32)
    pad = EP - E
    srcp = jnp.pad(src, (0, pad))
    dstp = jnp.pad(dst, (0, pad))
    typp = jnp.pad(typ, (0, pad), constant_values=R)
    zeros_h = jnp.zeros((ZROWS, LANES), jnp.float32)
    ones_h = jnp.ones((CH, LANES), jnp.float32)

    xb = jnp.transpose(x.reshape(N, NF, LANES), (1, 0, 2))
    S4, cnt3 = _make_sc(True)(xb, srcp, dstp, typp, zeros_h, ones_h)
    h = _tc_combine(x, S4.reshape(R, N, C), cnt3, W1, root1, b1, relu=True)

    hb = jnp.transpose(h.reshape(N, NF, LANES), (1, 0, 2))
    S4b = _make_sc(False)(hb, srcp, dstp, typp, zeros_h, ones_h)
    out = _tc_combine(h, S4b.reshape(R, N, C), cnt3, W2, root2, b2, relu=False)
    return out


# speedup vs baseline: 6.3659x; 1.3894x over previous
import functools

import jax
import jax.numpy as jnp
from jax import lax
from jax.experimental import pallas as pl
from jax.experimental.pallas import tpu as pltpu
from jax.experimental.pallas import tpu_sc as plsc

N = 10000
E = 160000
C = 128
R = 8
LANES = 16
NF = C // LANES
NS = 16
NC = 2
CH = 128
EPT = 10240
EP = EPT * NS
NCH = EPT // CH
KPAD = R * N
ACC_ROWS = 81920
RPT_Z = ACC_ROWS // NS
ZROWS = RPT_Z // 8
RPT = (R * N) // NS
BLK = 1000


def _sc_body(with_cnt, *refs):
    if with_cnt:
        (xb, srcp, dstp, typp, zeros_h, ones_h, s4, cnt3,
         src_v, kb, dstb, typb, rowb, onesb, zb, acc, sem) = refs
    else:
        (xb, srcp, dstp, typp, zeros_h, ones_h, s4,
         src_v, kb, dstb, typb, rowb, onesb, zb, acc, sem) = refs
        cnt3 = None

    c = lax.axis_index("c")
    s = lax.axis_index("s")
    e0 = s * EPT

    pltpu.sync_copy(srcp.at[pl.ds(e0, EPT)], src_v)
    pltpu.sync_copy(zeros_h, zb)
    pltpu.sync_copy(ones_h, onesb)

    def krow(j, carry):
        pltpu.sync_copy(dstp.at[pl.ds(e0 + j * CH, CH)], dstb)
        pltpu.sync_copy(typp.at[pl.ds(e0 + j * CH, CH)], typb)
        for u in range(CH // LANES):
            t16 = typb[pl.ds(u * LANES, LANES)]
            d16 = dstb[pl.ds(u * LANES, LANES)]
            kb[j, pl.ds(u * LANES, LANES)] = t16 * N + d16
        return carry
    lax.fori_loop(0, NCH, krow, 0)

    def zero_acc():
        def zbody(i, carry):
            pltpu.sync_copy(zb, acc.at[pl.ds(s * RPT_Z + i * ZROWS, ZROWS)])
            return carry
        lax.fori_loop(0, 8, zbody, 0)

    def scatter_pass(table):
        def chunk(j, carry):
            if table is not None:
                idx = src_v.at[pl.ds(j * CH, CH)]
                pltpu.async_copy(table.at[idx], rowb, sem).wait()
                rows = rowb
            else:
                rows = onesb
            pltpu.sync_copy(rows, acc.at[kb.at[j]], add=True)
            return carry
        lax.fori_loop(0, NCH, chunk, 0)

    def writeback(dst_ref, f):
        r = s // 2
        i0 = (s % 2) * RPT
        src_slice = acc.at[pl.ds(s * RPT, RPT)]
        if f is None:
            pltpu.sync_copy(src_slice, dst_ref.at[r, pl.ds(i0, RPT)])
        else:
            pltpu.sync_copy(
                src_slice,
                dst_ref.at[r, pl.ds(i0, RPT), pl.ds(f * LANES, LANES)])

    for cv in range(NC):
        @pl.when(c == cv)
        def _passes(cv=cv):
            for p in range(NF // NC):
                f = cv * (NF // NC) + p
                zero_acc()
                plsc.subcore_barrier()
                scatter_pass(xb.at[f])
                plsc.subcore_barrier()
                writeback(s4, f)
                plsc.subcore_barrier()

    if with_cnt:
        @pl.when(c == 0)
        def _cnt_pass():
            zero_acc()
            plsc.subcore_barrier()
            scatter_pass(None)
            plsc.subcore_barrier()
            writeback(cnt3, None)


def _make_sc(with_cnt):
    out_types = [jax.ShapeDtypeStruct((R, N, C), jnp.float32)]
    if with_cnt:
        out_types.append(jax.ShapeDtypeStruct((R, N, LANES), jnp.float32))
    scratch = [
        pltpu.VMEM((EPT,), jnp.int32),
        pltpu.VMEM((NCH, CH), jnp.int32),
        pltpu.VMEM((CH,), jnp.int32),
        pltpu.VMEM((CH,), jnp.int32),
        pltpu.VMEM((CH, LANES), jnp.float32),
        pltpu.VMEM((CH, LANES), jnp.float32),
        pltpu.VMEM((ZROWS, LANES), jnp.float32),
        pltpu.VMEM_SHARED((ACC_ROWS, LANES), jnp.float32),
        pltpu.SemaphoreType.DMA,
    ]
    mesh = plsc.VectorSubcoreMesh(core_axis_name="c", subcore_axis_name="s")
    return pl.kernel(
        functools.partial(_sc_body, with_cnt),
        out_type=tuple(out_types) if with_cnt else out_types[0],
        mesh=mesh,
        scratch_types=scratch,
        compiler_params=pltpu.CompilerParams(use_tc_tiling_on_sc=False),
    )


_make_sc = functools.lru_cache(maxsize=None)(_make_sc)


def _tc_body(relu, x_ref, s_ref, c_ref, w_ref, root_ref, b_ref, o_ref):
    acc = jnp.dot(x_ref[...], root_ref[...],
                  preferred_element_type=jnp.float32) + b_ref[...]
    for r in range(R):
        cr = c_ref[r][:, 0:1]
        inv = 1.0 / jnp.maximum(cr, 1.0)
        acc = acc + jnp.dot(s_ref[r] * inv, w_ref[r],
                            preferred_element_type=jnp.float32)
    o_ref[...] = jnp.maximum(acc, 0.0) if relu else acc


def _tc_combine(xin, S, cnt3, W, root, b, relu):
    return pl.pallas_call(
        functools.partial(_tc_body, relu),
        grid=(N // BLK,),
        in_specs=[
            pl.BlockSpec((BLK, C), lambda i: (i, 0)),
            pl.BlockSpec((R, BLK, C), lambda i: (0, i, 0)),
            pl.BlockSpec((R, BLK, LANES), lambda i: (0, i, 0)),
            pl.BlockSpec((R, C, C), lambda i: (0, 0, 0)),
            pl.BlockSpec((C, C), lambda i: (0, 0)),
            pl.BlockSpec((1, C), lambda i: (0, 0)),
        ],
        out_specs=pl.BlockSpec((BLK, C), lambda i: (i, 0)),
        out_shape=jax.ShapeDtypeStruct((N, C), jnp.float32),
    )(xin, S, cnt3, W, root, b.reshape(1, C))


def kernel(x, edge_index, edge_attr, W1, root1, b1, W2, root2, b2):
    src = edge_index[0].astype(jnp.int32)
    dst = edge_index[1].astype(jnp.int32)
    typ = edge_attr.astype(jnp.int32)
    pad = EP - E
    srcp = jnp.pad(src, (0, pad))
    dstp = jnp.pad(dst, (0, pad))
    typp = jnp.pad(typ, (0, pad), constant_values=R)
    zeros_h = jnp.zeros((ZROWS, LANES), jnp.float32)
    ones_h = jnp.ones((CH, LANES), jnp.float32)

    xb = jnp.transpose(x.reshape(N, NF, LANES), (1, 0, 2))
    S1, cnt3 = _make_sc(True)(xb, srcp, dstp, typp, zeros_h, ones_h)
    h = _tc_combine(x, S1, cnt3, W1, root1, b1, relu=True)

    hb = jnp.transpose(h.reshape(N, NF, LANES), (1, 0, 2))
    S2 = _make_sc(False)(hb, srcp, dstp, typp, zeros_h, ones_h)
    out = _tc_combine(h, S2, cnt3, W2, root2, b2, relu=False)
    return out


# speedup vs baseline: 7.6738x; 1.2055x over previous
import functools

import jax
import jax.numpy as jnp
from jax import lax
from jax.experimental import pallas as pl
from jax.experimental.pallas import tpu as pltpu
from jax.experimental.pallas import tpu_sc as plsc

N = 10000
E = 160000
C = 128
R = 8
LANES = 16
NF = C // LANES
NS = 16
NC = 2
CH = 128
EPT = 10240
EP = EPT * NS
NCH = EPT // CH
KPAD = R * N
ACC_ROWS = 81920
RPT_Z = ACC_ROWS // NS
ZROWS = RPT_Z // 8
RPT = (R * N) // NS
BLK = 1000


def _sc_body(with_cnt, *refs):
    if with_cnt:
        (xb, srcp, dstp, typp, zeros_h, ones_h, s4, cnt3,
         src_v, kb, dstb, typb, rowb, rowb2, onesb, zb, acc,
         sem, gsem, gsem2) = refs
    else:
        (xb, srcp, dstp, typp, zeros_h, ones_h, s4,
         src_v, kb, dstb, typb, rowb, rowb2, onesb, zb, acc,
         sem, gsem, gsem2) = refs
        cnt3 = None

    c = lax.axis_index("c")
    s = lax.axis_index("s")
    e0 = s * EPT

    pltpu.sync_copy(srcp.at[pl.ds(e0, EPT)], src_v)
    pltpu.sync_copy(zeros_h, zb)
    pltpu.sync_copy(ones_h, onesb)

    HALF = EPT // 2
    for half in range(2):
        pltpu.sync_copy(dstp.at[pl.ds(e0 + half * HALF, HALF)], dstb)
        pltpu.sync_copy(typp.at[pl.ds(e0 + half * HALF, HALF)], typb)

        def krow(jr, carry, half=half):
            j = half * (NCH // 2) + jr
            for u in range(CH // LANES):
                off = jr * CH + u * LANES
                t16 = typb[pl.ds(off, LANES)]
                d16 = dstb[pl.ds(off, LANES)]
                kb[j, pl.ds(u * LANES, LANES)] = t16 * N + d16
            return carry
        lax.fori_loop(0, NCH // 2, krow, 0)

    def zero_acc():
        def zbody(i, carry):
            pltpu.sync_copy(zb, acc.at[pl.ds(s * RPT_Z + i * ZROWS, ZROWS)])
            return carry
        lax.fori_loop(0, 8, zbody, 0)

    def scatter_pass(table):
        if table is None:
            def chunk(j, carry):
                pltpu.sync_copy(onesb, acc.at[kb.at[j]], add=True)
                return carry
            lax.fori_loop(0, NCH, chunk, 0)
            return

        def gather(j, buf, gs):
            idx = src_v.at[pl.ds(j * CH, CH)]
            return pltpu.async_copy(table.at[idx], buf, gs)

        def gwait(j, buf, gs):
            idx = src_v.at[pl.ds(j * CH, CH)]
            pltpu.make_async_copy(table.at[idx], buf, gs).wait()

        gather(0, rowb, gsem)

        def pair(jj, carry):
            j0 = 2 * jj
            j1 = j0 + 1
            gwait(j0, rowb, gsem)
            gather(j1, rowb2, gsem2)
            pltpu.sync_copy(rowb, acc.at[kb.at[j0]], add=True)
            gwait(j1, rowb2, gsem2)

            @pl.when(jj + 1 < NCH // 2)
            def _():
                gather(j0 + 2, rowb, gsem)
            pltpu.sync_copy(rowb2, acc.at[kb.at[j1]], add=True)
            return carry
        lax.fori_loop(0, NCH // 2, pair, 0)

    def writeback(dst_ref, f):
        r = s // 2
        i0 = (s % 2) * RPT
        src_slice = acc.at[pl.ds(s * RPT, RPT)]
        if f is None:
            pltpu.sync_copy(src_slice, dst_ref.at[r, pl.ds(i0, RPT)])
        else:
            pltpu.sync_copy(
                src_slice,
                dst_ref.at[r, pl.ds(i0, RPT), pl.ds(f * LANES, LANES)])

    for cv in range(NC):
        @pl.when(c == cv)
        def _passes(cv=cv):
            for p in range(NF // NC):
                f = cv * (NF // NC) + p
                zero_acc()
                plsc.subcore_barrier()
                scatter_pass(xb.at[f])
                plsc.subcore_barrier()
                writeback(s4, f)
                plsc.subcore_barrier()

    if with_cnt:
        @pl.when(c == 0)
        def _cnt_pass():
            zero_acc()
            plsc.subcore_barrier()
            scatter_pass(None)
            plsc.subcore_barrier()
            writeback(cnt3, None)


def _make_sc(with_cnt):
    out_types = [jax.ShapeDtypeStruct((R, N, C), jnp.float32)]
    if with_cnt:
        out_types.append(jax.ShapeDtypeStruct((R, N, LANES), jnp.float32))
    scratch = [
        pltpu.VMEM((EPT,), jnp.int32),
        pltpu.VMEM((NCH, CH), jnp.int32),
        pltpu.VMEM((EPT // 2,), jnp.int32),
        pltpu.VMEM((EPT // 2,), jnp.int32),
        pltpu.VMEM((CH, LANES), jnp.float32),
        pltpu.VMEM((CH, LANES), jnp.float32),
        pltpu.VMEM((CH, LANES), jnp.float32),
        pltpu.VMEM((ZROWS, LANES), jnp.float32),
        pltpu.VMEM_SHARED((ACC_ROWS, LANES), jnp.float32),
        pltpu.SemaphoreType.DMA,
        pltpu.SemaphoreType.DMA,
        pltpu.SemaphoreType.DMA,
    ]
    mesh = plsc.VectorSubcoreMesh(core_axis_name="c", subcore_axis_name="s")
    return pl.kernel(
        functools.partial(_sc_body, with_cnt),
        out_type=tuple(out_types) if with_cnt else out_types[0],
        mesh=mesh,
        scratch_types=scratch,
        compiler_params=pltpu.CompilerParams(use_tc_tiling_on_sc=False),
    )


_make_sc = functools.lru_cache(maxsize=None)(_make_sc)


def _tc_body(relu, x_ref, s_ref, c_ref, w_ref, root_ref, b_ref, o_ref):
    acc = jnp.dot(x_ref[...], root_ref[...],
                  preferred_element_type=jnp.float32) + b_ref[...]
    for r in range(R):
        cr = c_ref[r][:, 0:1]
        inv = 1.0 / jnp.maximum(cr, 1.0)
        acc = acc + jnp.dot(s_ref[r] * inv, w_ref[r],
                            preferred_element_type=jnp.float32)
    o_ref[...] = jnp.maximum(acc, 0.0) if relu else acc


def _tc_combine(xin, S, cnt3, W, root, b, relu):
    return pl.pallas_call(
        functools.partial(_tc_body, relu),
        grid=(N // BLK,),
        in_specs=[
            pl.BlockSpec((BLK, C), lambda i: (i, 0)),
            pl.BlockSpec((R, BLK, C), lambda i: (0, i, 0)),
            pl.BlockSpec((R, BLK, LANES), lambda i: (0, i, 0)),
            pl.BlockSpec((R, C, C), lambda i: (0, 0, 0)),
            pl.BlockSpec((C, C), lambda i: (0, 0)),
            pl.BlockSpec((1, C), lambda i: (0, 0)),
        ],
        out_specs=pl.BlockSpec((BLK, C), lambda i: (i, 0)),
        out_shape=jax.ShapeDtypeStruct((N, C), jnp.float32),
    )(xin, S, cnt3, W, root, b.reshape(1, C))


def kernel(x, edge_index, edge_attr, W1, root1, b1, W2, root2, b2):
    src = edge_index[0].astype(jnp.int32)
    dst = edge_index[1].astype(jnp.int32)
    typ = edge_attr.astype(jnp.int32)
    pad = EP - E
    srcp = jnp.pad(src, (0, pad))
    dstp = jnp.pad(dst, (0, pad))
    typp = jnp.pad(typ, (0, pad), constant_values=R)
    zeros_h = jnp.zeros((ZROWS, LANES), jnp.float32)
    ones_h = jnp.ones((CH, LANES), jnp.float32)

    xb = jnp.transpose(x.reshape(N, NF, LANES), (1, 0, 2))
    S1, cnt3 = _make_sc(True)(xb, srcp, dstp, typp, zeros_h, ones_h)
    h = _tc_combine(x, S1, cnt3, W1, root1, b1, relu=True)

    hb = jnp.transpose(h.reshape(N, NF, LANES), (1, 0, 2))
    S2 = _make_sc(False)(hb, srcp, dstp, typp, zeros_h, ones_h)
    out = _tc_combine(h, S2, cnt3, W2, root2, b2, relu=False)
    return out
